# baseline (speedup 1.0000x reference)
import functools

import jax
import jax.numpy as jnp
from jax import lax
from jax.experimental import pallas as pl
from jax.experimental.pallas import tpu as pltpu
from jax.experimental.pallas import tpu_sc as plsc

_TARGET_WEIGHT = 0.5

_ROWS = 16384
_COLS = 200
_N = _ROWS * _COLS

_SC_ROWS = 0
_TC_ROWS = _ROWS - _SC_ROWS
_TC_BLK = 2048
_TC_GRID = _TC_ROWS // _TC_BLK

_NC = 2
_NS = 16
_NW = _NC * _NS
_N_SC = _SC_ROWS * _COLS
_PER_W = _N_SC // _NW
_GROUPS = _PER_W // 64

_LN2 = 0.6931471805599453
_C5 = 0.030449
_C4 = -0.28382685
_C3 = 1.11609003
_C2 = -2.44002976
_C1 = 3.5140873
_C0 = -1.93675974 - 127.0 * _LN2


def _tc_body(p_ref, m_ref, out_ref):
    i = pl.program_id(0)
    p = p_ref[...]
    mf = m_ref[...].astype(jnp.float32)
    t = (1.0 - p) + mf * (p + p - 1.0)
    l2 = jnp.log2(t)
    l2n = jnp.maximum(l2, -1e30)
    s_t = jnp.sum(mf * l2)
    s_n = jnp.sum(l2n - mf * l2n)
    s_c = jnp.sum(mf)

    @pl.when(i == 0)
    def _init():
        out_ref[0, 0] = s_t
        out_ref[0, 1] = s_n
        out_ref[0, 2] = s_c

    @pl.when(i > 0)
    def _acc():
        out_ref[0, 0] += s_t
        out_ref[0, 1] += s_n
        out_ref[0, 2] += s_c


_tc_partials = pl.pallas_call(
    _tc_body,
    grid=(_TC_GRID,),
    in_specs=[
        pl.BlockSpec((_COLS, _TC_BLK), lambda i: (0, i)),
        pl.BlockSpec((_COLS, _TC_BLK), lambda i: (0, i)),
    ],
    out_specs=pl.BlockSpec((1, 3), lambda i: (0, 0),
                           memory_space=pltpu.SMEM),
    out_shape=jax.ShapeDtypeStruct((1, 3), jnp.float32),
)


def _log_f32(t):
    bits = plsc.bitcast(t, jnp.int32)
    raw_e = jnp.right_shift(bits, 23)
    mant = plsc.bitcast((bits & 0x7FFFFF) | 0x3F800000, jnp.float32)
    ef = raw_e.astype(jnp.float32)
    poly = _C5 * mant + _C4
    poly = poly * mant + _C3
    poly = poly * mant + _C2
    poly = poly * mant + _C1
    poly = poly * mant + _C0
    logt = ef * _LN2 + poly
    return jnp.where(t > 0.0, logt, jnp.float32(-jnp.inf))


def _sc_body(x_hbm, out_hbm, xbuf, vout, sem):
    wid = lax.axis_index("s") * _NC + lax.axis_index("c")
    base = pl.multiple_of(wid * _PER_W, _PER_W)

    cp = pltpu.make_async_copy(x_hbm.at[pl.ds(base, _PER_W)], xbuf, sem)
    cp.start()

    zf = jnp.zeros((16,), jnp.float32)

    def it(i, carry):
        acc_t, acc_n, cnt = carry
        off = i * 64
        nt, nn, nc = [], [], []
        for k in range(4):
            x = xbuf[pl.ds(off + 16 * k, 16)]
            pred = x >= 0.0
            t = jnp.abs(x)
            logt = _log_f32(t)
            nt.append(acc_t[k] + jnp.where(pred, logt, 0.0))
            nn.append(acc_n[k] + jnp.where(pred, 0.0, logt))
            nc.append(cnt[k] + jnp.where(pred, 1.0, 0.0))
        return tuple(nt), tuple(nn), tuple(nc)

    cp.wait()
    acc_t, acc_n, cnt = lax.fori_loop(
        0, _GROUPS, it,
        ((zf,) * 4, (zf,) * 4, (zf,) * 4), unroll=2)

    vout[0, :] = (acc_t[0] + acc_t[1]) + (acc_t[2] + acc_t[3])
    vout[1, :] = (acc_n[0] + acc_n[1]) + (acc_n[2] + acc_n[3])
    vout[2, :] = (cnt[0] + cnt[1]) + (cnt[2] + cnt[3])
    pltpu.sync_copy(vout, out_hbm.at[wid])


_sc_partials = None if not _SC_ROWS else functools.partial(
    pl.kernel,
    mesh=plsc.VectorSubcoreMesh(core_axis_name="c", subcore_axis_name="s"),
    out_type=jax.ShapeDtypeStruct((_NW, 3, 16), jnp.float32),
    compiler_params=pltpu.CompilerParams(needs_layout_passes=False),
    scratch_types=[
        pltpu.VMEM((_PER_W,), jnp.float32),
        pltpu.VMEM((3, 16), jnp.float32),
        pltpu.SemaphoreType.DMA,
    ],
)(_sc_body)


def kernel(prob_pred, mask_gt):
    if _SC_ROWS:
        p_tail = prob_pred[_TC_ROWS:]
        m_tail = mask_gt[_TC_ROWS:]
        x = jnp.where(m_tail, p_tail, p_tail - 1.0).reshape(_N_SC)
        sc = _sc_partials(x)
        sc_t = jnp.sum(sc[:, 0, :])
        sc_n = jnp.sum(sc[:, 1, :])
        sc_c = jnp.sum(sc[:, 2, :])
    else:
        sc_t = sc_n = sc_c = jnp.float32(0.0)

    tc = _tc_partials(prob_pred.T, mask_gt.T)

    sum_t = tc[0, 0] * jnp.float32(_LN2) + sc_t
    sum_n = tc[0, 1] * jnp.float32(_LN2) + sc_n
    n_t = tc[0, 2] + sc_c
    n_n = jnp.float32(_N) - n_t
    loss_t = -sum_t / n_t
    loss_n = -sum_n / n_n
    loss = _TARGET_WEIGHT * loss_t + (1.0 - _TARGET_WEIGHT) * loss_n
    return (loss, loss_t, loss_n)


# speedup vs baseline: 1.2026x; 1.2026x over previous
import functools

import jax
import jax.numpy as jnp
from jax import lax
from jax.experimental import pallas as pl
from jax.experimental.pallas import tpu as pltpu
from jax.experimental.pallas import tpu_sc as plsc

_TARGET_WEIGHT = 0.5

_ROWS = 16384
_COLS = 200
_N = _ROWS * _COLS

_SC_ROWS = 0
_TC_ROWS = _ROWS - _SC_ROWS
_TC_BLK = 2048
_TC_GRID = _TC_ROWS // _TC_BLK

_NC = 2
_NS = 16
_NW = _NC * _NS
_N_SC = _SC_ROWS * _COLS
_PER_W = _N_SC // _NW
_GROUPS = _PER_W // 64

_LN2 = 0.6931471805599453
_C5 = 0.030449
_C4 = -0.28382685
_C3 = 1.11609003
_C2 = -2.44002976
_C1 = 3.5140873
_C0 = -1.93675974 - 127.0 * _LN2


def _tc_body(p_ref, m_ref, out_ref):
    i = pl.program_id(0)
    p = p_ref[...]
    mf = m_ref[...].astype(jnp.float32)
    t = (1.0 - p) + mf * (p + p - 1.0)
    l2 = jnp.log2(t)
    l2n = jnp.maximum(l2, -1e30)
    s_t = jnp.sum(mf * l2)
    s_n = jnp.sum(l2n - mf * l2n)
    s_c = jnp.sum(mf)

    @pl.when(i == 0)
    def _init():
        out_ref[0, 0] = s_t
        out_ref[0, 1] = s_n
        out_ref[0, 2] = s_c

    @pl.when(i > 0)
    def _acc():
        out_ref[0, 0] += s_t
        out_ref[0, 1] += s_n
        out_ref[0, 2] += s_c


_tc_partials = pl.pallas_call(
    _tc_body,
    grid=(_TC_GRID,),
    in_specs=[
        pl.BlockSpec((_COLS, _TC_BLK), lambda i: (0, i)),
        pl.BlockSpec((_COLS, _TC_BLK), lambda i: (0, i)),
    ],
    out_specs=pl.BlockSpec((1, 3), lambda i: (0, 0),
                           memory_space=pltpu.SMEM),
    out_shape=jax.ShapeDtypeStruct((1, 3), jnp.float32),
)


def _log_f32(t):
    bits = plsc.bitcast(t, jnp.int32)
    raw_e = jnp.right_shift(bits, 23)
    mant = plsc.bitcast((bits & 0x7FFFFF) | 0x3F800000, jnp.float32)
    ef = raw_e.astype(jnp.float32)
    poly = _C5 * mant + _C4
    poly = poly * mant + _C3
    poly = poly * mant + _C2
    poly = poly * mant + _C1
    poly = poly * mant + _C0
    logt = ef * _LN2 + poly
    return jnp.where(t > 0.0, logt, jnp.float32(-jnp.inf))


def _sc_body(x_hbm, out_hbm, xbuf, vout, sem):
    wid = lax.axis_index("s") * _NC + lax.axis_index("c")
    base = pl.multiple_of(wid * _PER_W, _PER_W)

    cp = pltpu.make_async_copy(x_hbm.at[pl.ds(base, _PER_W)], xbuf, sem)
    cp.start()

    zf = jnp.zeros((16,), jnp.float32)

    def it(i, carry):
        acc_t, acc_n, cnt = carry
        off = i * 64
        nt, nn, nc = [], [], []
        for k in range(4):
            x = xbuf[pl.ds(off + 16 * k, 16)]
            pred = x >= 0.0
            t = jnp.abs(x)
            logt = _log_f32(t)
            nt.append(acc_t[k] + jnp.where(pred, logt, 0.0))
            nn.append(acc_n[k] + jnp.where(pred, 0.0, logt))
            nc.append(cnt[k] + jnp.where(pred, 1.0, 0.0))
        return tuple(nt), tuple(nn), tuple(nc)

    cp.wait()
    acc_t, acc_n, cnt = lax.fori_loop(
        0, _GROUPS, it,
        ((zf,) * 4, (zf,) * 4, (zf,) * 4), unroll=2)

    vout[0, :] = (acc_t[0] + acc_t[1]) + (acc_t[2] + acc_t[3])
    vout[1, :] = (acc_n[0] + acc_n[1]) + (acc_n[2] + acc_n[3])
    vout[2, :] = (cnt[0] + cnt[1]) + (cnt[2] + cnt[3])
    pltpu.sync_copy(vout, out_hbm.at[wid])


_sc_partials = None if not _SC_ROWS else functools.partial(
    pl.kernel,
    mesh=plsc.VectorSubcoreMesh(core_axis_name="c", subcore_axis_name="s"),
    out_type=jax.ShapeDtypeStruct((_NW, 3, 16), jnp.float32),
    compiler_params=pltpu.CompilerParams(needs_layout_passes=False),
    scratch_types=[
        pltpu.VMEM((_PER_W,), jnp.float32),
        pltpu.VMEM((3, 16), jnp.float32),
        pltpu.SemaphoreType.DMA,
    ],
)(_sc_body)


def kernel(prob_pred, mask_gt):
    if _SC_ROWS:
        p_tail = prob_pred[_TC_ROWS:]
        m_tail = mask_gt[_TC_ROWS:]
        x = jnp.where(m_tail, p_tail, p_tail - 1.0).reshape(_N_SC)
        sc = _sc_partials(x)
        sc_t = jnp.sum(sc[:, 0, :])
        sc_n = jnp.sum(sc[:, 1, :])
        sc_c = jnp.sum(sc[:, 2, :])
    else:
        sc_t = sc_n = sc_c = jnp.float32(0.0)

    tc = _tc_partials(prob_pred.T, mask_gt.view(jnp.uint8).T)

    sum_t = tc[0, 0] * jnp.float32(_LN2) + sc_t
    sum_n = tc[0, 1] * jnp.float32(_LN2) + sc_n
    n_t = tc[0, 2] + sc_c
    n_n = jnp.float32(_N) - n_t
    loss_t = -sum_t / n_t
    loss_n = -sum_n / n_n
    loss = _TARGET_WEIGHT * loss_t + (1.0 - _TARGET_WEIGHT) * loss_n
    return (loss, loss_t, loss_n)
